# Initial kernel scaffold; baseline (speedup 1.0000x reference)
import jax
import jax.numpy as jnp
from jax.experimental import pallas as pl


def kernel(x, edge_index, batch, W1, b1, p1, W2, b2, p2, W3, b3, p3, lw1, lb1, lw2, lb2):
    raise NotImplementedError("write your pallas kernel here")


# speedup vs baseline: 31.8701x; 31.8701x over previous
import functools

import jax
import jax.numpy as jnp
from jax import lax
from jax.experimental import pallas as pl
from jax.experimental.pallas import tpu as pltpu
from jax.experimental.pallas import tpu_sc as plsc

_N = 10000
_NP = 10240
_E = 320000
_H = 64
_NC = 2
_NS = 16
_NW = _NC * _NS
_EPT = _E // _NW
_RPT = _NP // _NS
_K1, _K2, _K3 = 8000, 6400, 5120

_mesh = plsc.VectorSubcoreMesh(
    core_axis_name="c", subcore_axis_name="s", num_cores=_NC, num_subcores=_NS
)

_DEG_CH = 2000


@functools.partial(
    pl.kernel,
    out_type=jax.ShapeDtypeStruct((_NC, _NP), jnp.float32),
    mesh=_mesh,
    scratch_types=[
        pltpu.VMEM((_DEG_CH,), jnp.int32),
        pltpu.VMEM((_DEG_CH,), jnp.int32),
        pltpu.VMEM((_DEG_CH,), jnp.float32),
        pltpu.VMEM((_RPT,), jnp.float32),
        pltpu.VMEM_SHARED((_NP,), jnp.float32),
        pltpu.VMEM_SHARED((_NP,), jnp.float32),
        pltpu.SemaphoreType.DMA,
    ],
    compiler_params=pltpu.CompilerParams(use_tc_tiling_on_sc=False),
)
def _sc_deg(src_hbm, dst_hbm, s_hbm, deg_out, srcv, dstv, updv, zrow, s_sh,
            deg_sh, sem):
    c = lax.axis_index("c")
    s = lax.axis_index("s")
    pltpu.sync_copy(s_hbm.at[pl.ds(s * _RPT, _RPT)], s_sh.at[pl.ds(s * _RPT, _RPT)])

    def _zb(i, carry):
        zrow[pl.ds(i * 16, 16)] = jnp.zeros((16,), jnp.float32)
        return carry

    lax.fori_loop(0, _RPT // 16, _zb, 0)
    pltpu.sync_copy(zrow, deg_sh.at[pl.ds(s * _RPT, _RPT)])
    plsc.subcore_barrier()

    base = (c * _NS + s) * _EPT
    for j in range(_EPT // _DEG_CH):
        pltpu.sync_copy(src_hbm.at[pl.ds(base + j * _DEG_CH, _DEG_CH)], srcv)
        pltpu.sync_copy(dst_hbm.at[pl.ds(base + j * _DEG_CH, _DEG_CH)], dstv)
        pltpu.async_copy(s_sh.at[srcv], updv, sem).wait()
        pltpu.sync_copy(updv, deg_sh.at[dstv], add=True)

    plsc.subcore_barrier()
    pltpu.sync_copy(deg_sh.at[pl.ds(s * _RPT, _RPT)],
                    deg_out.at[c, pl.ds(s * _RPT, _RPT)])


_AGG_CH = 1000


@functools.partial(
    pl.kernel,
    out_type=jax.ShapeDtypeStruct((_NC, _NP, _H), jnp.float32),
    mesh=_mesh,
    scratch_types=[
        pltpu.VMEM((_AGG_CH,), jnp.int32),
        pltpu.VMEM((_AGG_CH,), jnp.int32),
        pltpu.VMEM((_AGG_CH, _H), jnp.float32),
        pltpu.VMEM_SHARED((_NP, _H), jnp.float32),
        pltpu.SemaphoreType.DMA,
    ],
    compiler_params=pltpu.CompilerParams(use_tc_tiling_on_sc=False),
)
def _sc_agg(src_hbm, dst_hbm, y_hbm, z_hbm, raw_out, srcv, dstv, rows,
            acc_sh, sem):
    c = lax.axis_index("c")
    s = lax.axis_index("s")
    pltpu.sync_copy(z_hbm.at[pl.ds(s * _RPT, _RPT)], acc_sh.at[pl.ds(s * _RPT, _RPT)])
    plsc.subcore_barrier()

    base = (c * _NS + s) * _EPT
    for j in range(_EPT // _AGG_CH):
        pltpu.sync_copy(src_hbm.at[pl.ds(base + j * _AGG_CH, _AGG_CH)], srcv)
        pltpu.sync_copy(dst_hbm.at[pl.ds(base + j * _AGG_CH, _AGG_CH)], dstv)
        pltpu.async_copy(y_hbm.at[srcv], rows, sem).wait()
        pltpu.sync_copy(rows, acc_sh.at[dstv], add=True)

    plsc.subcore_barrier()
    pltpu.sync_copy(acc_sh.at[pl.ds(s * _RPT, _RPT)],
                    raw_out.at[c, pl.ds(s * _RPT, _RPT)])


def _tc_pre(x, W, s_col, d0, d1):
    def body(x_ref, w_ref, s_ref, d0_ref, d1_ref, xw_ref, y_ref, g_ref):
        xw = jnp.dot(x_ref[...], w_ref[...], preferred_element_type=jnp.float32)
        deg = d0_ref[...] + d1_ref[...] + 1.0
        g = s_ref[...] * lax.rsqrt(deg)
        xw_ref[...] = xw
        y_ref[...] = g * xw
        g_ref[...] = g

    return pl.pallas_call(
        body,
        out_shape=[
            jax.ShapeDtypeStruct((_NP, _H), jnp.float32),
            jax.ShapeDtypeStruct((_NP, _H), jnp.float32),
            jax.ShapeDtypeStruct((_NP, 1), jnp.float32),
        ],
    )(x, W, s_col, d0, d1)


def _tc_post(raw, xw, g_col, b_row, p_row, r_prev, k):
    def body(raw_ref, xw_ref, g_ref, b_ref, p_ref, r_ref, xn_ref, sel_ref, rn_ref):
        g = g_ref[...]
        raw01 = raw_ref[0, :, :] + raw_ref[1, :, :]
        h = g * raw01 + (g * g) * xw_ref[...] + b_ref[...]
        h = jnp.maximum(h, 0.0)

        pv = p_ref[...]
        pn = jnp.sqrt(jnp.sum(pv * pv)) + 1e-16
        sc_row = lax.dot_general(pv, h, (((1,), (1,)), ((), ())),
                                 preferred_element_type=jnp.float32) / pn
        sc_row = jnp.tanh(sc_row)

        g_row = jnp.reshape(g, (1, _NP))
        key = jnp.where(g_row > 0, sc_row, -jnp.inf)
        bu = lax.bitcast_convert_type(key, jnp.uint32)
        ku = jnp.where(bu >= jnp.uint32(0x80000000), ~bu,
                       bu | jnp.uint32(0x80000000))

        def bit_body(i, t):
            bit = jnp.uint32(1) << (jnp.uint32(31) - i.astype(jnp.uint32))
            t_try = t | bit
            cnt = jnp.sum((ku >= t_try).astype(jnp.int32))
            return jnp.where(cnt >= k, t_try, t)

        t = lax.fori_loop(0, 32, bit_body, jnp.uint32(0))
        gt = ku > t
        eq = ku == t
        need = k - jnp.sum(gt.astype(jnp.int32))
        idx = lax.broadcasted_iota(jnp.int32, (1, _NP), 1)

        def jb(i, T):
            T_try = T | (jnp.int32(1) << (jnp.int32(13) - i))
            cnt = jnp.sum((eq & (idx <= T_try - 1)).astype(jnp.int32))
            return jnp.where(cnt < need, T_try, T)

        T = lax.fori_loop(0, 14, jb, jnp.int32(0))
        sel_row = gt | (eq & (idx <= T) & (need > 0))
        gate_row = jnp.where(sel_row, sc_row, 0.0)
        gate_col = jnp.reshape(gate_row, (_NP, 1))
        sel_col = jnp.reshape(sel_row.astype(jnp.float32), (_NP, 1))

        xn = h * gate_col
        xn_ref[...] = xn
        sel_ref[...] = sel_col
        rmax = jnp.max(jnp.where(sel_col > 0, xn, -jnp.inf), axis=0,
                       keepdims=True)
        rmean = jnp.sum(xn, axis=0, keepdims=True) * (1.0 / k)
        rn_ref[...] = r_ref[...] + jnp.concatenate([rmax, rmean], axis=1)

    return pl.pallas_call(
        body,
        out_shape=[
            jax.ShapeDtypeStruct((_NP, _H), jnp.float32),
            jax.ShapeDtypeStruct((_NP, 1), jnp.float32),
            jax.ShapeDtypeStruct((1, 2 * _H), jnp.float32),
        ],
    )(raw, xw, g_col, b_row, p_row, r_prev)


def _tc_fin(r, lw1, lb1_row, lw2, lb2_row):
    def body(r_ref, w1_ref, b1_ref, w2_ref, b2_ref, o_ref):
        t1 = jnp.dot(r_ref[...], w1_ref[...], preferred_element_type=jnp.float32)
        t1 = jnp.maximum(t1 + b1_ref[...], 0.0)
        o_ref[...] = jnp.dot(t1, w2_ref[...],
                             preferred_element_type=jnp.float32) + b2_ref[...]

    return pl.pallas_call(
        body,
        out_shape=jax.ShapeDtypeStruct((1, 16), jnp.float32),
    )(r, lw1, lb1_row, lw2, lb2_row)


def kernel(x, edge_index, batch, W1, b1, p1, W2, b2, p2, W3, b3, p3,
           lw1, lb1, lw2, lb2):
    src = edge_index[0]
    dst = edge_index[1]
    x_cur = jnp.pad(x, ((0, _NP - _N), (0, 0)))
    s_col = (jnp.arange(_NP) < _N).astype(jnp.float32)[:, None]
    r = jnp.zeros((1, 2 * _H), jnp.float32)
    zeros_tab = jnp.zeros((_NP, _H), jnp.float32)

    for (W, b, p, k) in ((W1, b1, p1, _K1), (W2, b2, p2, _K2), (W3, b3, p3, _K3)):
        s_vec = s_col[:, 0]
        deg_parts = _sc_deg(src, dst, s_vec)
        d0 = deg_parts[0][:, None]
        d1 = deg_parts[1][:, None]
        xw, y, g_col = _tc_pre(x_cur, W, s_col, d0, d1)
        raw = _sc_agg(src, dst, y, zeros_tab)
        x_cur, s_col, r = _tc_post(raw, xw, g_col, b[None, :], p[None, :], r, k)

    return _tc_fin(r, lw1, lb1[None, :], lw2, lb2[None, :])


# speedup vs baseline: 50.2848x; 1.5778x over previous
import functools

import jax
import jax.numpy as jnp
from jax import lax
from jax.experimental import pallas as pl
from jax.experimental.pallas import tpu as pltpu
from jax.experimental.pallas import tpu_sc as plsc

_N = 10000
_NP = 10240
_E = 320000
_H = 64
_NC = 2
_NS = 16
_NW = _NC * _NS
_EPT = _E // _NW
_RPT = _NP // _NS
_K1, _K2, _K3 = 8000, 6400, 5120

_mesh = plsc.VectorSubcoreMesh(
    core_axis_name="c", subcore_axis_name="s", num_cores=_NC, num_subcores=_NS
)

_DEG_CH = 2000


@functools.partial(
    pl.kernel,
    out_type=jax.ShapeDtypeStruct((_NC, _NP), jnp.float32),
    mesh=_mesh,
    scratch_types=[
        pltpu.VMEM((_DEG_CH,), jnp.int32),
        pltpu.VMEM((_DEG_CH,), jnp.int32),
        pltpu.VMEM((_DEG_CH,), jnp.float32),
        pltpu.VMEM((_RPT,), jnp.float32),
        pltpu.VMEM_SHARED((_NP,), jnp.float32),
        pltpu.VMEM_SHARED((_NP,), jnp.float32),
        pltpu.SemaphoreType.DMA,
    ],
    compiler_params=pltpu.CompilerParams(use_tc_tiling_on_sc=False),
)
def _sc_deg(src_hbm, dst_hbm, s_hbm, deg_out, srcv, dstv, updv, zrow, s_sh,
            deg_sh, sem):
    c = lax.axis_index("c")
    s = lax.axis_index("s")
    pltpu.sync_copy(s_hbm.at[pl.ds(s * _RPT, _RPT)], s_sh.at[pl.ds(s * _RPT, _RPT)])

    def _zb(i, carry):
        zrow[pl.ds(i * 16, 16)] = jnp.zeros((16,), jnp.float32)
        return carry

    lax.fori_loop(0, _RPT // 16, _zb, 0)
    pltpu.sync_copy(zrow, deg_sh.at[pl.ds(s * _RPT, _RPT)])
    plsc.subcore_barrier()

    base = (c * _NS + s) * _EPT
    for j in range(_EPT // _DEG_CH):
        pltpu.sync_copy(src_hbm.at[pl.ds(base + j * _DEG_CH, _DEG_CH)], srcv)
        pltpu.sync_copy(dst_hbm.at[pl.ds(base + j * _DEG_CH, _DEG_CH)], dstv)
        pltpu.async_copy(s_sh.at[srcv], updv, sem).wait()
        pltpu.sync_copy(updv, deg_sh.at[dstv], add=True)

    plsc.subcore_barrier()
    pltpu.sync_copy(deg_sh.at[pl.ds(s * _RPT, _RPT)],
                    deg_out.at[c, pl.ds(s * _RPT, _RPT)])


_AGG_CH = 400
_AGG_NCH = _EPT // _AGG_CH


@functools.partial(
    pl.kernel,
    out_type=jax.ShapeDtypeStruct((_NC, _NP, _H), jnp.float32),
    mesh=_mesh,
    scratch_types=[
        pltpu.VMEM((2, _AGG_CH), jnp.int32),
        pltpu.VMEM((2, _AGG_CH), jnp.int32),
        pltpu.VMEM((2, _AGG_CH, _H), jnp.float32),
        pltpu.VMEM_SHARED((_NP, _H), jnp.float32),
        pltpu.SemaphoreType.DMA,
        pltpu.SemaphoreType.DMA,
    ],
    compiler_params=pltpu.CompilerParams(use_tc_tiling_on_sc=False),
)
def _sc_agg(src_hbm, dst_hbm, y_hbm, z_hbm, raw_out, srcv2, dstv2, rows2,
            acc_sh, gsem, ssem):
    c = lax.axis_index("c")
    s = lax.axis_index("s")
    pltpu.sync_copy(z_hbm.at[pl.ds(s * _RPT, _RPT)], acc_sh.at[pl.ds(s * _RPT, _RPT)])
    plsc.subcore_barrier()

    base = (c * _NS + s) * _EPT

    def _issue_gather(j, buf):
        pltpu.sync_copy(src_hbm.at[pl.ds(base + j * _AGG_CH, _AGG_CH)],
                        srcv2.at[buf])
        pltpu.sync_copy(dst_hbm.at[pl.ds(base + j * _AGG_CH, _AGG_CH)],
                        dstv2.at[buf])
        return pltpu.async_copy(y_hbm.at[srcv2.at[buf]], rows2.at[buf], gsem)

    g = _issue_gather(0, 0)
    scat = None
    for j in range(_AGG_NCH):
        buf = j % 2
        g.wait()
        if scat is not None:
            scat.wait()
        if j + 1 < _AGG_NCH:
            g = _issue_gather(j + 1, 1 - buf)
        scat = pltpu.async_copy(rows2.at[buf], acc_sh.at[dstv2.at[buf]], ssem,
                                add=True)
    scat.wait()

    plsc.subcore_barrier()
    pltpu.sync_copy(acc_sh.at[pl.ds(s * _RPT, _RPT)],
                    raw_out.at[c, pl.ds(s * _RPT, _RPT)])


def _tc_pre(x, W, s_col, d_col):
    def body(x_ref, w_ref, s_ref, d_ref, y_ref, g_ref):
        xw = jnp.dot(x_ref[...], w_ref[...], preferred_element_type=jnp.float32)
        g = s_ref[...] * lax.rsqrt(d_ref[...] + 1.0)
        y_ref[...] = g * xw
        g_ref[...] = g

    return pl.pallas_call(
        body,
        out_shape=[
            jax.ShapeDtypeStruct((_NP, _H), jnp.float32),
            jax.ShapeDtypeStruct((_NP, 1), jnp.float32),
        ],
    )(x, W, s_col, d_col)


_R8, _C8 = 8, _NP // 8


def _radix_select_level(bucket, live, kk, from_top):
    cnts = [jnp.sum(jnp.where(bucket == b, 1.0, 0.0)) for b in range(16)]
    order = range(15, -1, -1) if from_top else range(16)
    acc = jnp.float32(0.0)
    B = jnp.int32(0)
    fcnt = jnp.float32(0.0)
    for b in order:
        flag = (acc < kk) & (acc + cnts[b] >= kk)
        B = jnp.where(flag, jnp.int32(b), B)
        fcnt = jnp.where(flag, acc, fcnt)
        acc = acc + cnts[b]
    if from_top:
        outside = jnp.where(bucket > B, live, 0.0)
    else:
        outside = jnp.where((bucket < B) & (bucket >= 0), live, 0.0)
    live_next = jnp.where(bucket == B, live, 0.0)
    return outside, live_next, kk - fcnt


def _tc_post(rawpy, g_col, s_row, b_row, p_row, k):
    def body(raw_ref, g_ref, s_ref, b_ref, p_ref, h_ref, sel_ref, gate_ref):
        g = g_ref[...]
        h = g * raw_ref[...] + b_ref[...]
        h = jnp.maximum(h, 0.0)

        pv = p_ref[...]
        pn = jnp.sqrt(jnp.sum(pv * pv)) + 1e-16
        sc_row = jnp.tanh(lax.dot_general(pv, h, (((1,), (1,)), ((), ())),
                          preferred_element_type=jnp.float32) / pn)

        key = jnp.where(s_ref[...] > 0, sc_row, -jnp.inf)
        bu = lax.bitcast_convert_type(jnp.reshape(key, (_R8, _C8)), jnp.uint32)
        ku = jnp.where(bu >= jnp.uint32(0x80000000), ~bu,
                       bu | jnp.uint32(0x80000000))

        def key_level(lvl, carry):
            live, gt, kk = carry
            sh = jnp.uint32(28) - jnp.uint32(4) * lvl.astype(jnp.uint32)
            bucket = ((ku >> sh) & jnp.uint32(0xF)).astype(jnp.int32)
            bucket = jnp.where(live > 0, bucket, -1)
            out_l, live, kk = _radix_select_level(bucket, live, kk, True)
            return live, gt + out_l, kk

        live, gt, kk = lax.fori_loop(
            0, 8, key_level,
            (jnp.ones((_R8, _C8), jnp.float32),
             jnp.zeros((_R8, _C8), jnp.float32), jnp.float32(k)))
        idx = (lax.broadcasted_iota(jnp.int32, (_R8, _C8), 0) * _C8 +
               lax.broadcasted_iota(jnp.int32, (_R8, _C8), 1))

        def idx_level(lvl, carry):
            live, lt, kk = carry
            sh = jnp.int32(12) - jnp.int32(4) * lvl
            bucket = (idx >> sh) & 0xF
            bucket = jnp.where(live > 0, bucket, -1)
            out_l, live, kk = _radix_select_level(bucket, live, kk, False)
            return live, lt + out_l, kk

        live, lt, kk = lax.fori_loop(
            0, 4, idx_level,
            (live, jnp.zeros((_R8, _C8), jnp.float32), kk))
        sel_row = jnp.reshape(gt + lt + live, (1, _NP))
        sel_ref[...] = sel_row
        gate_ref[...] = jnp.where(sel_row > 0, sc_row, 0.0)
        h_ref[...] = h

    return pl.pallas_call(
        body,
        out_shape=[
            jax.ShapeDtypeStruct((_NP, _H), jnp.float32),
            jax.ShapeDtypeStruct((1, _NP), jnp.float32),
            jax.ShapeDtypeStruct((1, _NP), jnp.float32),
        ],
    )(rawpy, g_col, s_row, b_row, p_row)


def _tc_pre2(h, gate_col, s_col, d_col, W, r_prev, k_prev):
    def body(h_ref, gate_ref, s_ref, d_ref, w_ref, r_ref, y_ref, g_ref, rn_ref):
        s = s_ref[...]
        xn = h_ref[...] * gate_ref[...]
        rmax = jnp.max(jnp.where(s > 0, xn, -jnp.inf), axis=0, keepdims=True)
        rmean = jnp.sum(xn, axis=0, keepdims=True) * (1.0 / k_prev)
        rn_ref[...] = r_ref[...] + jnp.concatenate([rmax, rmean], axis=1)
        xw = jnp.dot(xn, w_ref[...], preferred_element_type=jnp.float32)
        g = s * lax.rsqrt(d_ref[...] + 1.0)
        y_ref[...] = g * xw
        g_ref[...] = g

    return pl.pallas_call(
        body,
        out_shape=[
            jax.ShapeDtypeStruct((_NP, _H), jnp.float32),
            jax.ShapeDtypeStruct((_NP, 1), jnp.float32),
            jax.ShapeDtypeStruct((1, 2 * _H), jnp.float32),
        ],
    )(h, gate_col, s_col, d_col, W, r_prev)


def _tc_fin(h, gate_col, s_col, r, lw1, lb1_row, lw2, lb2_row, k_prev):
    def body(h_ref, gate_ref, s_ref, r_ref, w1_ref, b1_ref, w2_ref, b2_ref,
             o_ref):
        s = s_ref[...]
        xn = h_ref[...] * gate_ref[...]
        rmax = jnp.max(jnp.where(s > 0, xn, -jnp.inf), axis=0, keepdims=True)
        rmean = jnp.sum(xn, axis=0, keepdims=True) * (1.0 / k_prev)
        rr = r_ref[...] + jnp.concatenate([rmax, rmean], axis=1)
        t1 = jnp.dot(rr, w1_ref[...], preferred_element_type=jnp.float32)
        t1 = jnp.maximum(t1 + b1_ref[...], 0.0)
        o_ref[...] = jnp.dot(t1, w2_ref[...],
                             preferred_element_type=jnp.float32) + b2_ref[...]

    return pl.pallas_call(
        body,
        out_shape=jax.ShapeDtypeStruct((1, 16), jnp.float32),
    )(h, gate_col, s_col, r, lw1, lb1_row, lw2, lb2_row)


def kernel(x, edge_index, batch, W1, b1, p1, W2, b2, p2, W3, b3, p3,
           lw1, lb1, lw2, lb2):
    src = edge_index[0]
    dst = edge_index[1]
    x_pad = jnp.pad(x, ((0, _NP - _N), (0, 0)))
    s_col = (jnp.arange(_NP) < _N).astype(jnp.float32)[:, None]
    r = jnp.zeros((1, 2 * _H), jnp.float32)
    zeros_tab = jnp.zeros((_NP, _H), jnp.float32)

    layers = ((W1, b1, p1, _K1), (W2, b2, p2, _K2), (W3, b3, p3, _K3))
    h = None
    gate_col = None
    k_prev = None
    s_row = jnp.reshape(s_col, (1, _NP))
    for li, (W, b, p, k) in enumerate(layers):
        s_vec = jnp.reshape(s_col, (_NP,))
        deg_parts = _sc_deg(src, dst, s_vec)
        d_col = (deg_parts[0] + deg_parts[1])[:, None]
        if li == 0:
            y, g_col = _tc_pre(x_pad, W, s_col, d_col)
        else:
            y, g_col, r = _tc_pre2(h, gate_col, s_col, d_col, W, r, k_prev)
        raw = _sc_agg(src, dst, y, zeros_tab)
        rawpy = raw[0] + raw[1] + y
        h, sel_row, gate_row = _tc_post(rawpy, g_col, s_row, b[None, :],
                                        p[None, :], k)
        s_row = sel_row
        s_col = jnp.reshape(sel_row, (_NP, 1))
        gate_col = jnp.reshape(gate_row, (_NP, 1))
        k_prev = k

    return _tc_fin(h, gate_col, s_col, r, lw1, lb1[None, :], lw2, lb2[None, :],
                   k_prev)


# speedup vs baseline: 58.0319x; 1.1541x over previous
import functools

import jax
import jax.numpy as jnp
from jax import lax
from jax.experimental import pallas as pl
from jax.experimental.pallas import tpu as pltpu
from jax.experimental.pallas import tpu_sc as plsc

_N = 10000
_NP = 10240
_E = 320000
_H = 64
_NC = 2
_NS = 16
_NW = _NC * _NS
_EPT = _E // _NW
_RPT = _NP // _NS
_K1, _K2, _K3 = 8000, 6400, 5120

_mesh = plsc.VectorSubcoreMesh(
    core_axis_name="c", subcore_axis_name="s", num_cores=_NC, num_subcores=_NS
)

_DEG_CH = 2000


@functools.partial(
    pl.kernel,
    out_type=jax.ShapeDtypeStruct((_NC, _NP), jnp.float32),
    mesh=_mesh,
    scratch_types=[
        pltpu.VMEM((_DEG_CH,), jnp.int32),
        pltpu.VMEM((_DEG_CH,), jnp.int32),
        pltpu.VMEM((_DEG_CH,), jnp.float32),
        pltpu.VMEM((_RPT,), jnp.float32),
        pltpu.VMEM_SHARED((_NP,), jnp.float32),
        pltpu.VMEM_SHARED((_NP,), jnp.float32),
        pltpu.SemaphoreType.DMA,
    ],
    compiler_params=pltpu.CompilerParams(use_tc_tiling_on_sc=False),
)
def _sc_deg(src_hbm, dst_hbm, s_hbm, deg_out, srcv, dstv, updv, zrow, s_sh,
            deg_sh, sem):
    c = lax.axis_index("c")
    s = lax.axis_index("s")
    pltpu.sync_copy(s_hbm.at[pl.ds(s * _RPT, _RPT)], s_sh.at[pl.ds(s * _RPT, _RPT)])

    def _zb(i, carry):
        zrow[pl.ds(i * 16, 16)] = jnp.zeros((16,), jnp.float32)
        return carry

    lax.fori_loop(0, _RPT // 16, _zb, 0)
    pltpu.sync_copy(zrow, deg_sh.at[pl.ds(s * _RPT, _RPT)])
    plsc.subcore_barrier()

    base = (c * _NS + s) * _EPT
    for j in range(_EPT // _DEG_CH):
        pltpu.sync_copy(src_hbm.at[pl.ds(base + j * _DEG_CH, _DEG_CH)], srcv)
        pltpu.sync_copy(dst_hbm.at[pl.ds(base + j * _DEG_CH, _DEG_CH)], dstv)
        pltpu.async_copy(s_sh.at[srcv], updv, sem).wait()
        pltpu.sync_copy(updv, deg_sh.at[dstv], add=True)

    plsc.subcore_barrier()
    pltpu.sync_copy(deg_sh.at[pl.ds(s * _RPT, _RPT)],
                    deg_out.at[c, pl.ds(s * _RPT, _RPT)])


_AGG_CH = 400
_AGG_NCH = _EPT // _AGG_CH


@functools.partial(
    pl.kernel,
    out_type=jax.ShapeDtypeStruct((_NC, _NP, _H), jnp.float32),
    mesh=_mesh,
    scratch_types=[
        pltpu.VMEM((_AGG_NCH, _AGG_CH), jnp.int32),
        pltpu.VMEM((_AGG_NCH, _AGG_CH), jnp.int32),
        pltpu.VMEM((2, _AGG_CH, _H), jnp.float32),
        pltpu.VMEM_SHARED((_NP, _H), jnp.float32),
        pltpu.SemaphoreType.DMA,
        pltpu.SemaphoreType.DMA,
    ],
    compiler_params=pltpu.CompilerParams(use_tc_tiling_on_sc=False),
)
def _sc_agg(src3_hbm, dst3_hbm, y_hbm, z_hbm, raw_out, srcb, dstb, rows2,
            acc_sh, gsem, ssem):
    c = lax.axis_index("c")
    s = lax.axis_index("s")
    w = c * _NS + s
    pltpu.sync_copy(src3_hbm.at[w], srcb)
    pltpu.sync_copy(dst3_hbm.at[w], dstb)
    pltpu.sync_copy(z_hbm.at[pl.ds(s * _RPT, _RPT)], acc_sh.at[pl.ds(s * _RPT, _RPT)])
    plsc.subcore_barrier()

    g = pltpu.async_copy(y_hbm.at[srcb.at[0]], rows2.at[0], gsem)
    scat = None
    for j in range(_AGG_NCH):
        buf = j % 2
        g.wait()
        if scat is not None:
            scat.wait()
        if j + 1 < _AGG_NCH:
            g = pltpu.async_copy(y_hbm.at[srcb.at[j + 1]], rows2.at[1 - buf],
                                 gsem)
        scat = pltpu.async_copy(rows2.at[buf], acc_sh.at[dstb.at[j]], ssem,
                                add=True)
    scat.wait()

    plsc.subcore_barrier()
    pltpu.sync_copy(acc_sh.at[pl.ds(s * _RPT, _RPT)],
                    raw_out.at[c, pl.ds(s * _RPT, _RPT)])


def _tc_pre(x, W, s_col, d_col):
    def body(x_ref, w_ref, s_ref, d_ref, y_ref, g_ref):
        xw = jnp.dot(x_ref[...], w_ref[...], preferred_element_type=jnp.float32)
        g = s_ref[...] * lax.rsqrt(d_ref[...] + 1.0)
        y_ref[...] = g * xw
        g_ref[...] = g

    return pl.pallas_call(
        body,
        out_shape=[
            jax.ShapeDtypeStruct((_NP, _H), jnp.float32),
            jax.ShapeDtypeStruct((_NP, 1), jnp.float32),
        ],
    )(x, W, s_col, d_col)


_R8, _C8 = 8, _NP // 8


def _radix_select_level(bucket, live, kk, from_top):
    cnts = [jnp.sum(jnp.where(bucket == b, 1.0, 0.0)) for b in range(16)]
    order = range(15, -1, -1) if from_top else range(16)
    acc = jnp.float32(0.0)
    B = jnp.int32(0)
    fcnt = jnp.float32(0.0)
    for b in order:
        flag = (acc < kk) & (acc + cnts[b] >= kk)
        B = jnp.where(flag, jnp.int32(b), B)
        fcnt = jnp.where(flag, acc, fcnt)
        acc = acc + cnts[b]
    if from_top:
        outside = jnp.where(bucket > B, live, 0.0)
    else:
        outside = jnp.where((bucket < B) & (bucket >= 0), live, 0.0)
    live_next = jnp.where(bucket == B, live, 0.0)
    return outside, live_next, kk - fcnt


def _tc_post(rawpy, g_col, s_row, b_row, p_row, k):
    def body(raw_ref, g_ref, s_ref, b_ref, p_ref, h_ref, sel_ref, gate_ref):
        g = g_ref[...]
        h = g * raw_ref[...] + b_ref[...]
        h = jnp.maximum(h, 0.0)

        pv = p_ref[...]
        pn = jnp.sqrt(jnp.sum(pv * pv)) + 1e-16
        sc_row = jnp.tanh(lax.dot_general(pv, h, (((1,), (1,)), ((), ())),
                          preferred_element_type=jnp.float32) / pn)

        key = jnp.where(s_ref[...] > 0, sc_row, -jnp.inf)
        bu = lax.bitcast_convert_type(jnp.reshape(key, (_R8, _C8)), jnp.uint32)
        ku = jnp.where(bu >= jnp.uint32(0x80000000), ~bu,
                       bu | jnp.uint32(0x80000000))

        def key_level(lvl, carry):
            live, gt, kk = carry
            sh = jnp.uint32(28) - jnp.uint32(4) * lvl.astype(jnp.uint32)
            bucket = ((ku >> sh) & jnp.uint32(0xF)).astype(jnp.int32)
            bucket = jnp.where(live > 0, bucket, -1)
            out_l, live, kk = _radix_select_level(bucket, live, kk, True)
            return live, gt + out_l, kk

        live, gt, kk = lax.fori_loop(
            0, 8, key_level,
            (jnp.ones((_R8, _C8), jnp.float32),
             jnp.zeros((_R8, _C8), jnp.float32), jnp.float32(k)))
        idx = (lax.broadcasted_iota(jnp.int32, (_R8, _C8), 0) * _C8 +
               lax.broadcasted_iota(jnp.int32, (_R8, _C8), 1))

        def idx_level(lvl, carry):
            live, lt, kk = carry
            sh = jnp.int32(12) - jnp.int32(4) * lvl
            bucket = (idx >> sh) & 0xF
            bucket = jnp.where(live > 0, bucket, -1)
            out_l, live, kk = _radix_select_level(bucket, live, kk, False)
            return live, lt + out_l, kk

        live, lt, kk = lax.fori_loop(
            0, 4, idx_level,
            (live, jnp.zeros((_R8, _C8), jnp.float32), kk))
        sel_row = jnp.reshape(gt + lt + live, (1, _NP))
        sel_ref[...] = sel_row
        gate_ref[...] = jnp.where(sel_row > 0, sc_row, 0.0)
        h_ref[...] = h

    return pl.pallas_call(
        body,
        out_shape=[
            jax.ShapeDtypeStruct((_NP, _H), jnp.float32),
            jax.ShapeDtypeStruct((1, _NP), jnp.float32),
            jax.ShapeDtypeStruct((1, _NP), jnp.float32),
        ],
    )(rawpy, g_col, s_row, b_row, p_row)


def _tc_pre2(h, gate_col, s_col, d_col, W, r_prev, k_prev):
    def body(h_ref, gate_ref, s_ref, d_ref, w_ref, r_ref, y_ref, g_ref, rn_ref):
        s = s_ref[...]
        xn = h_ref[...] * gate_ref[...]
        rmax = jnp.max(jnp.where(s > 0, xn, -jnp.inf), axis=0, keepdims=True)
        rmean = jnp.sum(xn, axis=0, keepdims=True) * (1.0 / k_prev)
        rn_ref[...] = r_ref[...] + jnp.concatenate([rmax, rmean], axis=1)
        xw = jnp.dot(xn, w_ref[...], preferred_element_type=jnp.float32)
        g = s * lax.rsqrt(d_ref[...] + 1.0)
        y_ref[...] = g * xw
        g_ref[...] = g

    return pl.pallas_call(
        body,
        out_shape=[
            jax.ShapeDtypeStruct((_NP, _H), jnp.float32),
            jax.ShapeDtypeStruct((_NP, 1), jnp.float32),
            jax.ShapeDtypeStruct((1, 2 * _H), jnp.float32),
        ],
    )(h, gate_col, s_col, d_col, W, r_prev)


def _tc_fin(h, gate_col, s_col, r, lw1, lb1_row, lw2, lb2_row, k_prev):
    def body(h_ref, gate_ref, s_ref, r_ref, w1_ref, b1_ref, w2_ref, b2_ref,
             o_ref):
        s = s_ref[...]
        xn = h_ref[...] * gate_ref[...]
        rmax = jnp.max(jnp.where(s > 0, xn, -jnp.inf), axis=0, keepdims=True)
        rmean = jnp.sum(xn, axis=0, keepdims=True) * (1.0 / k_prev)
        rr = r_ref[...] + jnp.concatenate([rmax, rmean], axis=1)
        t1 = jnp.dot(rr, w1_ref[...], preferred_element_type=jnp.float32)
        t1 = jnp.maximum(t1 + b1_ref[...], 0.0)
        o_ref[...] = jnp.dot(t1, w2_ref[...],
                             preferred_element_type=jnp.float32) + b2_ref[...]

    return pl.pallas_call(
        body,
        out_shape=jax.ShapeDtypeStruct((1, 16), jnp.float32),
    )(h, gate_col, s_col, r, lw1, lb1_row, lw2, lb2_row)


def kernel(x, edge_index, batch, W1, b1, p1, W2, b2, p2, W3, b3, p3,
           lw1, lb1, lw2, lb2):
    src = edge_index[0]
    dst = edge_index[1]
    src3 = jnp.reshape(src, (_NW, _AGG_NCH, _AGG_CH))
    dst3 = jnp.reshape(dst, (_NW, _AGG_NCH, _AGG_CH))
    x_pad = jnp.pad(x, ((0, _NP - _N), (0, 0)))
    s_col = (jnp.arange(_NP) < _N).astype(jnp.float32)[:, None]
    r = jnp.zeros((1, 2 * _H), jnp.float32)
    zeros_tab = jnp.zeros((_NP, _H), jnp.float32)

    layers = ((W1, b1, p1, _K1), (W2, b2, p2, _K2), (W3, b3, p3, _K3))
    h = None
    gate_col = None
    k_prev = None
    s_row = jnp.reshape(s_col, (1, _NP))
    for li, (W, b, p, k) in enumerate(layers):
        s_vec = jnp.reshape(s_col, (_NP,))
        deg_parts = _sc_deg(src, dst, s_vec)
        d_col = (deg_parts[0] + deg_parts[1])[:, None]
        if li == 0:
            y, g_col = _tc_pre(x_pad, W, s_col, d_col)
        else:
            y, g_col, r = _tc_pre2(h, gate_col, s_col, d_col, W, r, k_prev)
        raw = _sc_agg(src3, dst3, y, zeros_tab)
        rawpy = raw[0] + raw[1] + y
        h, sel_row, gate_row = _tc_post(rawpy, g_col, s_row, b[None, :],
                                        p[None, :], k)
        s_row = sel_row
        s_col = jnp.reshape(sel_row, (_NP, 1))
        gate_col = jnp.reshape(gate_row, (_NP, 1))
        k_prev = k

    return _tc_fin(h, gate_col, s_col, r, lw1, lb1[None, :], lw2, lb2[None, :],
                   k_prev)


# speedup vs baseline: 59.4036x; 1.0236x over previous
import functools

import jax
import jax.numpy as jnp
from jax import lax
from jax.experimental import pallas as pl
from jax.experimental.pallas import tpu as pltpu
from jax.experimental.pallas import tpu_sc as plsc

_N = 10000
_NP = 10240
_E = 320000
_H = 64
_NC = 2
_NS = 16
_NW = _NC * _NS
_EPT = _E // _NW
_RPT = _NP // _NS
_K1, _K2, _K3 = 8000, 6400, 5120

_mesh = plsc.VectorSubcoreMesh(
    core_axis_name="c", subcore_axis_name="s", num_cores=_NC, num_subcores=_NS
)

_DEG_CH = 2000


@functools.partial(
    pl.kernel,
    out_type=jax.ShapeDtypeStruct((_NC, _NP), jnp.float32),
    mesh=_mesh,
    scratch_types=[
        pltpu.VMEM((_DEG_CH,), jnp.int32),
        pltpu.VMEM((_DEG_CH,), jnp.int32),
        pltpu.VMEM((_DEG_CH,), jnp.float32),
        pltpu.VMEM((_RPT,), jnp.float32),
        pltpu.VMEM_SHARED((_NP,), jnp.float32),
        pltpu.VMEM_SHARED((_NP,), jnp.float32),
        pltpu.SemaphoreType.DMA,
    ],
    compiler_params=pltpu.CompilerParams(use_tc_tiling_on_sc=False),
)
def _sc_deg(ei_hbm, s_hbm, deg_out, srcv, dstv, updv, zrow, s_sh,
            deg_sh, sem):
    c = lax.axis_index("c")
    s = lax.axis_index("s")
    pltpu.sync_copy(s_hbm.at[pl.ds(s * _RPT, _RPT)], s_sh.at[pl.ds(s * _RPT, _RPT)])

    def _zb(i, carry):
        zrow[pl.ds(i * 16, 16)] = jnp.zeros((16,), jnp.float32)
        return carry

    lax.fori_loop(0, _RPT // 16, _zb, 0)
    pltpu.sync_copy(zrow, deg_sh.at[pl.ds(s * _RPT, _RPT)])
    plsc.subcore_barrier()

    base = (c * _NS + s) * _EPT
    for j in range(_EPT // _DEG_CH):
        pltpu.sync_copy(ei_hbm.at[0, pl.ds(base + j * _DEG_CH, _DEG_CH)], srcv)
        pltpu.sync_copy(ei_hbm.at[1, pl.ds(base + j * _DEG_CH, _DEG_CH)], dstv)
        pltpu.async_copy(s_sh.at[srcv], updv, sem).wait()
        pltpu.sync_copy(updv, deg_sh.at[dstv], add=True)

    plsc.subcore_barrier()
    pltpu.sync_copy(deg_sh.at[pl.ds(s * _RPT, _RPT)],
                    deg_out.at[c, pl.ds(s * _RPT, _RPT)])


_AGG_CH = 400
_AGG_NCH = _EPT // _AGG_CH


@functools.partial(
    pl.kernel,
    out_type=jax.ShapeDtypeStruct((_NC, _NP, _H), jnp.float32),
    mesh=_mesh,
    scratch_types=[
        pltpu.VMEM((_AGG_NCH, _AGG_CH), jnp.int32),
        pltpu.VMEM((_AGG_NCH, _AGG_CH), jnp.int32),
        pltpu.VMEM((2, _AGG_CH, _H), jnp.float32),
        pltpu.VMEM_SHARED((_NP, _H), jnp.float32),
        pltpu.SemaphoreType.DMA,
        pltpu.SemaphoreType.DMA,
    ],
    compiler_params=pltpu.CompilerParams(use_tc_tiling_on_sc=False),
)
def _sc_agg(ei4_hbm, y_hbm, z_hbm, raw_out, srcb, dstb, rows2,
            acc_sh, gsem, ssem):
    c = lax.axis_index("c")
    s = lax.axis_index("s")
    w = c * _NS + s
    pltpu.sync_copy(ei4_hbm.at[0, w], srcb)
    pltpu.sync_copy(ei4_hbm.at[1, w], dstb)
    pltpu.sync_copy(z_hbm.at[pl.ds(s * _RPT, _RPT)], acc_sh.at[pl.ds(s * _RPT, _RPT)])
    plsc.subcore_barrier()

    g = pltpu.async_copy(y_hbm.at[srcb.at[0]], rows2.at[0], gsem)
    scat = None
    for j in range(_AGG_NCH):
        buf = j % 2
        g.wait()
        if scat is not None:
            scat.wait()
        if j + 1 < _AGG_NCH:
            g = pltpu.async_copy(y_hbm.at[srcb.at[j + 1]], rows2.at[1 - buf],
                                 gsem)
        scat = pltpu.async_copy(rows2.at[buf], acc_sh.at[dstb.at[j]], ssem,
                                add=True)
    scat.wait()

    plsc.subcore_barrier()
    pltpu.sync_copy(acc_sh.at[pl.ds(s * _RPT, _RPT)],
                    raw_out.at[c, pl.ds(s * _RPT, _RPT)])


def _tc_pre(x, W, s_col, d_col):
    def body(x_ref, w_ref, s_ref, d_ref, y_ref, g_ref):
        xw = jnp.dot(x_ref[...], w_ref[...], preferred_element_type=jnp.float32)
        g = s_ref[...] * lax.rsqrt(d_ref[...] + 1.0)
        y_ref[...] = g * xw
        g_ref[...] = g

    return pl.pallas_call(
        body,
        out_shape=[
            jax.ShapeDtypeStruct((_NP, _H), jnp.float32),
            jax.ShapeDtypeStruct((_NP, 1), jnp.float32),
        ],
    )(x, W, s_col, d_col)


_R8, _C8 = 8, _NP // 8


def _radix_select_level(bucket, live, kk, from_top):
    cnts = [jnp.sum(jnp.where(bucket == b, 1.0, 0.0)) for b in range(16)]
    order = range(15, -1, -1) if from_top else range(16)
    acc = jnp.float32(0.0)
    B = jnp.int32(0)
    fcnt = jnp.float32(0.0)
    for b in order:
        flag = (acc < kk) & (acc + cnts[b] >= kk)
        B = jnp.where(flag, jnp.int32(b), B)
        fcnt = jnp.where(flag, acc, fcnt)
        acc = acc + cnts[b]
    if from_top:
        outside = jnp.where(bucket > B, live, 0.0)
    else:
        outside = jnp.where((bucket < B) & (bucket >= 0), live, 0.0)
    live_next = jnp.where(bucket == B, live, 0.0)
    return outside, live_next, kk - fcnt


def _tc_post(rawpy, g_col, s_row, b_row, p_row, k):
    def body(raw_ref, g_ref, s_ref, b_ref, p_ref, h_ref, sel_ref, gate_ref):
        g = g_ref[...]
        h = g * raw_ref[...] + b_ref[...]
        h = jnp.maximum(h, 0.0)

        pv = p_ref[...]
        pn = jnp.sqrt(jnp.sum(pv * pv)) + 1e-16
        sc_row = jnp.tanh(lax.dot_general(pv, h, (((1,), (1,)), ((), ())),
                          preferred_element_type=jnp.float32) / pn)

        key = jnp.where(s_ref[...] > 0, sc_row, -jnp.inf)
        bu = lax.bitcast_convert_type(jnp.reshape(key, (_R8, _C8)), jnp.uint32)
        ku = jnp.where(bu >= jnp.uint32(0x80000000), ~bu,
                       bu | jnp.uint32(0x80000000))

        def key_level(lvl, carry):
            live, gt, kk = carry
            sh = jnp.uint32(28) - jnp.uint32(4) * lvl.astype(jnp.uint32)
            bucket = ((ku >> sh) & jnp.uint32(0xF)).astype(jnp.int32)
            bucket = jnp.where(live > 0, bucket, -1)
            out_l, live, kk = _radix_select_level(bucket, live, kk, True)
            return live, gt + out_l, kk

        live, gt, kk = lax.fori_loop(
            0, 8, key_level,
            (jnp.ones((_R8, _C8), jnp.float32),
             jnp.zeros((_R8, _C8), jnp.float32), jnp.float32(k)))
        idx = (lax.broadcasted_iota(jnp.int32, (_R8, _C8), 0) * _C8 +
               lax.broadcasted_iota(jnp.int32, (_R8, _C8), 1))

        def idx_level(lvl, carry):
            live, lt, kk = carry
            sh = jnp.int32(12) - jnp.int32(4) * lvl
            bucket = (idx >> sh) & 0xF
            bucket = jnp.where(live > 0, bucket, -1)
            out_l, live, kk = _radix_select_level(bucket, live, kk, False)
            return live, lt + out_l, kk

        live, lt, kk = lax.fori_loop(
            0, 4, idx_level,
            (live, jnp.zeros((_R8, _C8), jnp.float32), kk))
        sel_row = jnp.reshape(gt + lt + live, (1, _NP))
        sel_ref[...] = sel_row
        gate_ref[...] = jnp.where(sel_row > 0, sc_row, 0.0)
        h_ref[...] = h

    return pl.pallas_call(
        body,
        out_shape=[
            jax.ShapeDtypeStruct((_NP, _H), jnp.float32),
            jax.ShapeDtypeStruct((1, _NP), jnp.float32),
            jax.ShapeDtypeStruct((1, _NP), jnp.float32),
        ],
    )(rawpy, g_col, s_row, b_row, p_row)


def _tc_pre2(h, gate_col, s_col, d_col, W, r_prev, k_prev):
    def body(h_ref, gate_ref, s_ref, d_ref, w_ref, r_ref, y_ref, g_ref, rn_ref):
        s = s_ref[...]
        xn = h_ref[...] * gate_ref[...]
        rmax = jnp.max(jnp.where(s > 0, xn, -jnp.inf), axis=0, keepdims=True)
        rmean = jnp.sum(xn, axis=0, keepdims=True) * (1.0 / k_prev)
        rn_ref[...] = r_ref[...] + jnp.concatenate([rmax, rmean], axis=1)
        xw = jnp.dot(xn, w_ref[...], preferred_element_type=jnp.float32)
        g = s * lax.rsqrt(d_ref[...] + 1.0)
        y_ref[...] = g * xw
        g_ref[...] = g

    return pl.pallas_call(
        body,
        out_shape=[
            jax.ShapeDtypeStruct((_NP, _H), jnp.float32),
            jax.ShapeDtypeStruct((_NP, 1), jnp.float32),
            jax.ShapeDtypeStruct((1, 2 * _H), jnp.float32),
        ],
    )(h, gate_col, s_col, d_col, W, r_prev)


def _tc_fin(h, gate_col, s_col, r, lw1, lb1_row, lw2, lb2_row, k_prev):
    def body(h_ref, gate_ref, s_ref, r_ref, w1_ref, b1_ref, w2_ref, b2_ref,
             o_ref):
        s = s_ref[...]
        xn = h_ref[...] * gate_ref[...]
        rmax = jnp.max(jnp.where(s > 0, xn, -jnp.inf), axis=0, keepdims=True)
        rmean = jnp.sum(xn, axis=0, keepdims=True) * (1.0 / k_prev)
        rr = r_ref[...] + jnp.concatenate([rmax, rmean], axis=1)
        t1 = jnp.dot(rr, w1_ref[...], preferred_element_type=jnp.float32)
        t1 = jnp.maximum(t1 + b1_ref[...], 0.0)
        o_ref[...] = jnp.dot(t1, w2_ref[...],
                             preferred_element_type=jnp.float32) + b2_ref[...]

    return pl.pallas_call(
        body,
        out_shape=jax.ShapeDtypeStruct((1, 16), jnp.float32),
    )(h, gate_col, s_col, r, lw1, lb1_row, lw2, lb2_row)


def kernel(x, edge_index, batch, W1, b1, p1, W2, b2, p2, W3, b3, p3,
           lw1, lb1, lw2, lb2):
    ei4 = jnp.reshape(edge_index, (2, _NW, _AGG_NCH, _AGG_CH))
    x_pad = jnp.pad(x, ((0, _NP - _N), (0, 0)))
    s_col = (jnp.arange(_NP) < _N).astype(jnp.float32)[:, None]
    r = jnp.zeros((1, 2 * _H), jnp.float32)
    zeros_tab = jnp.zeros((_NP, _H), jnp.float32)

    layers = ((W1, b1, p1, _K1), (W2, b2, p2, _K2), (W3, b3, p3, _K3))
    h = None
    gate_col = None
    k_prev = None
    s_row = jnp.reshape(s_col, (1, _NP))
    for li, (W, b, p, k) in enumerate(layers):
        s_vec = jnp.reshape(s_col, (_NP,))
        deg_parts = _sc_deg(edge_index, s_vec)
        d_col = (deg_parts[0] + deg_parts[1])[:, None]
        if li == 0:
            y, g_col = _tc_pre(x_pad, W, s_col, d_col)
        else:
            y, g_col, r = _tc_pre2(h, gate_col, s_col, d_col, W, r, k_prev)
        raw = _sc_agg(ei4, y, zeros_tab)
        rawpy = raw[0] + raw[1] + y
        h, sel_row, gate_row = _tc_post(rawpy, g_col, s_row, b[None, :],
                                        p[None, :], k)
        s_row = sel_row
        s_col = jnp.reshape(sel_row, (_NP, 1))
        gate_col = jnp.reshape(gate_row, (_NP, 1))
        k_prev = k

    return _tc_fin(h, gate_col, s_col, r, lw1, lb1[None, :], lw2, lb2[None, :],
                   k_prev)
